# baseline (speedup 1.0000x reference)
import functools

import jax
import jax.numpy as jnp
from jax import lax
from jax.experimental import pallas as pl
from jax.experimental.pallas import tpu as pltpu
from jax.experimental.pallas import tpu_sc as plsc

NSEG = 1024
D = 128
NROWS = 100000
CHUNK = 128
NFULL = NROWS // CHUNK
TAIL = NROWS - NFULL * CHUNK
NCORES = 2
NSUB = 16
NW = NCORES * NSUB
BASE_SPAN = NFULL // NW
EXTRA = NFULL - BASE_SPAN * NW
MAX_SPAN = BASE_SPAN + 1
SEG_PER_SUB = NSEG // NSUB


def _sc_partials(H, idx1d, idx_tail, zeros_stripe):
    mesh = plsc.VectorSubcoreMesh(core_axis_name="c", subcore_axis_name="s")

    @functools.partial(
        pl.kernel,
        mesh=mesh,
        out_type=jax.ShapeDtypeStruct((NCORES, NSEG, D), jnp.float32),
        scratch_types=[
            pltpu.VMEM((2, CHUNK, D), jnp.float32),
            pltpu.VMEM((2, CHUNK), jnp.int32),
            pltpu.VMEM((TAIL, D), jnp.float32),
            pltpu.VMEM((TAIL,), jnp.int32),
            pltpu.VMEM_SHARED((NSEG, D), jnp.float32),
            pltpu.SemaphoreType.DMA((2,)),
            pltpu.SemaphoreType.DMA((2,)),
        ],
    )
    def k(h_hbm, idx_hbm, idxt_hbm, z_hbm, out_hbm,
          rows_v, idx_v, rows_t, idx_t, acc_sh, sem, sem_sc):
        c = lax.axis_index("c")
        s = lax.axis_index("s")
        t = s * NCORES + c

        pltpu.sync_copy(z_hbm, rows_v.at[0, pl.ds(0, SEG_PER_SUB)])
        pltpu.sync_copy(rows_v.at[0, pl.ds(0, SEG_PER_SUB)],
                        acc_sh.at[pl.ds(s * SEG_PER_SUB, SEG_PER_SUB)])

        start = t * BASE_SPAN + jnp.minimum(t, EXTRA)
        n_t = jnp.where(t < EXTRA, MAX_SPAN, BASE_SPAN)
        plsc.subcore_barrier()

        def issue(j, b):
            pltpu.async_copy(
                h_hbm.at[pl.ds((start + j) * CHUNK, CHUNK)],
                rows_v.at[b], sem.at[b])
            pltpu.async_copy(
                idx_hbm.at[pl.ds((start + j) * CHUNK, CHUNK)],
                idx_v.at[b], sem.at[b])

        def wait_scatter(b):
            pltpu.make_async_copy(
                h_hbm.at[pl.ds(0, CHUNK)], rows_v.at[b], sem_sc.at[b]).wait()

        issue(0, 0)

        def body(j, carry):
            b = lax.rem(j, 2)
            nb = 1 - b

            @pl.when(j + 1 < n_t)
            def _():
                @pl.when(j >= 1)
                def _():
                    wait_scatter(nb)
                issue(j + 1, nb)

            pltpu.make_async_copy(
                h_hbm.at[pl.ds(0, CHUNK)], rows_v.at[b], sem.at[b]).wait()
            pltpu.make_async_copy(
                idx_hbm.at[pl.ds(0, CHUNK)], idx_v.at[b], sem.at[b]).wait()
            pltpu.async_copy(rows_v.at[b], acc_sh.at[idx_v.at[b]],
                             sem_sc.at[b], add=True)
            return carry

        lax.fori_loop(0, n_t, body, 0)
        wait_scatter(lax.rem(n_t - 1, 2))
        wait_scatter(lax.rem(n_t, 2))

        @pl.when(t == NW - 1)
        def _():
            base = NFULL * CHUNK
            pltpu.sync_copy(h_hbm.at[pl.ds(base, TAIL)], rows_t)
            pltpu.sync_copy(idxt_hbm, idx_t)
            pltpu.sync_copy(rows_t, acc_sh.at[idx_t], add=True)

        plsc.subcore_barrier()

        pltpu.sync_copy(acc_sh.at[pl.ds(s * SEG_PER_SUB, SEG_PER_SUB)],
                        rows_v.at[0, pl.ds(0, SEG_PER_SUB)])
        pltpu.sync_copy(rows_v.at[0, pl.ds(0, SEG_PER_SUB)],
                        out_hbm.at[c, pl.ds(s * SEG_PER_SUB, SEG_PER_SUB)])

    return k(H, idx1d, idx_tail, zeros_stripe)


def _combine(partials):
    def body(p_ref, o_ref):
        o_ref[...] = p_ref[0] + p_ref[1]

    return pl.pallas_call(
        body,
        out_shape=jax.ShapeDtypeStruct((NSEG, D), jnp.float32),
    )(partials)


def kernel(H, batch_idx):
    idx = batch_idx.astype(jnp.int32)
    idx_tail = idx[NFULL * CHUNK:]
    zeros_stripe = jnp.zeros((SEG_PER_SUB, D), jnp.float32)
    partials = _sc_partials(H, idx, idx_tail, zeros_stripe)
    return _combine(partials)


# speedup vs baseline: 1.0571x; 1.0571x over previous
import functools

import jax
import jax.numpy as jnp
from jax import lax
from jax.experimental import pallas as pl
from jax.experimental.pallas import tpu as pltpu
from jax.experimental.pallas import tpu_sc as plsc

NSEG = 1024
D = 128
NROWS = 100000
CHUNK = 128
NFULL = NROWS // CHUNK
TAIL = NROWS - NFULL * CHUNK
NCORES = 2
NSUB = 16
NW = NCORES * NSUB
BASE_SPAN = NFULL // NW
EXTRA = NFULL - BASE_SPAN * NW
MAX_SPAN = BASE_SPAN + 1
SEG_PER_SUB = NSEG // NSUB
NBUF = 4


def _sc_partials(H, idx1d, idx_tail, zeros_stripe):
    mesh = plsc.VectorSubcoreMesh(core_axis_name="c", subcore_axis_name="s")

    @functools.partial(
        pl.kernel,
        mesh=mesh,
        out_type=jax.ShapeDtypeStruct((NCORES, NSEG, D), jnp.float32),
        scratch_types=[
            pltpu.VMEM((NBUF, CHUNK, D), jnp.float32),
            pltpu.VMEM((NBUF, CHUNK), jnp.int32),
            pltpu.VMEM((TAIL, D), jnp.float32),
            pltpu.VMEM((TAIL,), jnp.int32),
            pltpu.VMEM_SHARED((NSEG, D), jnp.float32),
            pltpu.SemaphoreType.DMA((NBUF,)),
            pltpu.SemaphoreType.DMA((NBUF,)),
        ],
    )
    def k(h_hbm, idx_hbm, idxt_hbm, z_hbm, out_hbm,
          rows_v, idx_v, rows_t, idx_t, acc_sh, sem, sem_sc):
        c = lax.axis_index("c")
        s = lax.axis_index("s")
        t = s * NCORES + c

        pltpu.sync_copy(z_hbm, rows_v.at[0, pl.ds(0, SEG_PER_SUB)])
        pltpu.sync_copy(rows_v.at[0, pl.ds(0, SEG_PER_SUB)],
                        acc_sh.at[pl.ds(s * SEG_PER_SUB, SEG_PER_SUB)])

        start = t * BASE_SPAN + jnp.minimum(t, EXTRA)
        n_t = jnp.where(t < EXTRA, MAX_SPAN, BASE_SPAN)
        plsc.subcore_barrier()

        def issue(j, b):
            pltpu.async_copy(
                h_hbm.at[pl.ds((start + j) * CHUNK, CHUNK)],
                rows_v.at[b], sem.at[b])
            pltpu.async_copy(
                idx_hbm.at[pl.ds((start + j) * CHUNK, CHUNK)],
                idx_v.at[b], sem.at[b])

        def wait_scatter(b):
            pltpu.make_async_copy(
                h_hbm.at[pl.ds(0, CHUNK)], rows_v.at[b], sem_sc.at[b]).wait()

        issue(0, 0)
        issue(1, 1)

        def body(j, carry):
            b = lax.rem(j, NBUF)

            @pl.when(j + 2 < n_t)
            def _():
                issue(j + 2, lax.rem(j + 2, NBUF))

            pltpu.make_async_copy(
                h_hbm.at[pl.ds(0, CHUNK)], rows_v.at[b], sem.at[b]).wait()
            pltpu.make_async_copy(
                idx_hbm.at[pl.ds(0, CHUNK)], idx_v.at[b], sem.at[b]).wait()

            @pl.when(j >= 1)
            def _():
                wait_scatter(lax.rem(j - 1, NBUF))

            pltpu.async_copy(rows_v.at[b], acc_sh.at[idx_v.at[b]],
                             sem_sc.at[b], add=True)
            return carry

        lax.fori_loop(0, n_t, body, 0)
        wait_scatter(lax.rem(n_t - 1, NBUF))

        @pl.when(t == NW - 1)
        def _():
            base = NFULL * CHUNK
            pltpu.sync_copy(h_hbm.at[pl.ds(base, TAIL)], rows_t)
            pltpu.sync_copy(idxt_hbm, idx_t)
            pltpu.sync_copy(rows_t, acc_sh.at[idx_t], add=True)

        plsc.subcore_barrier()

        pltpu.sync_copy(acc_sh.at[pl.ds(s * SEG_PER_SUB, SEG_PER_SUB)],
                        rows_v.at[0, pl.ds(0, SEG_PER_SUB)])
        pltpu.sync_copy(rows_v.at[0, pl.ds(0, SEG_PER_SUB)],
                        out_hbm.at[c, pl.ds(s * SEG_PER_SUB, SEG_PER_SUB)])

    return k(H, idx1d, idx_tail, zeros_stripe)


def _combine(partials):
    def body(p_ref, o_ref):
        o_ref[...] = p_ref[0] + p_ref[1]

    return pl.pallas_call(
        body,
        out_shape=jax.ShapeDtypeStruct((NSEG, D), jnp.float32),
    )(partials)


def kernel(H, batch_idx):
    idx = batch_idx.astype(jnp.int32)
    idx_tail = idx[NFULL * CHUNK:]
    zeros_stripe = jnp.zeros((SEG_PER_SUB, D), jnp.float32)
    partials = _sc_partials(H, idx, idx_tail, zeros_stripe)
    return _combine(partials)


# speedup vs baseline: 1.1018x; 1.0423x over previous
import functools

import jax
import jax.numpy as jnp
from jax import lax
from jax.experimental import pallas as pl
from jax.experimental.pallas import tpu as pltpu
from jax.experimental.pallas import tpu_sc as plsc

NSEG = 1024
D = 128
NROWS = 100000
CHUNK = 128
NFULL = NROWS // CHUNK
TAIL = NROWS - NFULL * CHUNK
NCORES = 2
NSUB = 16
NW = NCORES * NSUB
BASE_SPAN = NFULL // NW
EXTRA = NFULL - BASE_SPAN * NW
MAX_SPAN = BASE_SPAN + 1
SEG_PER_SUB = NSEG // NSUB
NBUF = 4


def _sc_partials(H, idx1d, idx_tail):
    mesh = plsc.VectorSubcoreMesh(core_axis_name="c", subcore_axis_name="s")

    @functools.partial(
        pl.kernel,
        mesh=mesh,
        out_type=jax.ShapeDtypeStruct((NCORES, NSEG, D), jnp.float32),
        scratch_types=[
            pltpu.VMEM((NBUF, CHUNK, D), jnp.float32),
            pltpu.VMEM((NBUF, CHUNK), jnp.int32),
            pltpu.VMEM((TAIL, D), jnp.float32),
            pltpu.VMEM((TAIL,), jnp.int32),
            pltpu.VMEM_SHARED((NSEG, D), jnp.float32),
            pltpu.SemaphoreType.DMA((NBUF,)),
            pltpu.SemaphoreType.DMA((NBUF,)),
        ],
    )
    def k(h_hbm, idx_hbm, idxt_hbm, out_hbm,
          rows_v, idx_v, rows_t, idx_t, acc_sh, sem, sem_sc):
        c = lax.axis_index("c")
        s = lax.axis_index("s")
        t = s * NCORES + c

        zvec = jnp.zeros((16,), jnp.float32)

        def zrow(r, carry):
            for jj in range(D // 16):
                rows_v[0, r, pl.ds(jj * 16, 16)] = zvec
            return carry

        lax.fori_loop(0, SEG_PER_SUB, zrow, 0)
        pltpu.sync_copy(rows_v.at[0, pl.ds(0, SEG_PER_SUB)],
                        acc_sh.at[pl.ds(s * SEG_PER_SUB, SEG_PER_SUB)])

        start = t * BASE_SPAN + jnp.minimum(t, EXTRA)
        n_t = jnp.where(t < EXTRA, MAX_SPAN, BASE_SPAN)
        plsc.subcore_barrier()

        def issue(j, b):
            pltpu.async_copy(
                h_hbm.at[pl.ds((start + j) * CHUNK, CHUNK)],
                rows_v.at[b], sem.at[b])
            pltpu.async_copy(
                idx_hbm.at[pl.ds((start + j) * CHUNK, CHUNK)],
                idx_v.at[b], sem.at[b])

        def wait_scatter(b):
            pltpu.make_async_copy(
                h_hbm.at[pl.ds(0, CHUNK)], rows_v.at[b], sem_sc.at[b]).wait()

        issue(0, 0)
        issue(1, 1)

        def body(j, carry):
            b = lax.rem(j, NBUF)

            @pl.when(j + 2 < n_t)
            def _():
                issue(j + 2, lax.rem(j + 2, NBUF))

            pltpu.make_async_copy(
                h_hbm.at[pl.ds(0, CHUNK)], rows_v.at[b], sem.at[b]).wait()
            pltpu.make_async_copy(
                idx_hbm.at[pl.ds(0, CHUNK)], idx_v.at[b], sem.at[b]).wait()

            @pl.when(j >= 1)
            def _():
                wait_scatter(lax.rem(j - 1, NBUF))

            pltpu.async_copy(rows_v.at[b], acc_sh.at[idx_v.at[b]],
                             sem_sc.at[b], add=True)
            return carry

        lax.fori_loop(0, n_t, body, 0)
        wait_scatter(lax.rem(n_t - 1, NBUF))

        @pl.when(t == NW - 1)
        def _():
            base = NFULL * CHUNK
            pltpu.sync_copy(h_hbm.at[pl.ds(base, TAIL)], rows_t)
            pltpu.sync_copy(idxt_hbm, idx_t)
            pltpu.sync_copy(rows_t, acc_sh.at[idx_t], add=True)

        plsc.subcore_barrier()

        pltpu.sync_copy(acc_sh.at[pl.ds(s * SEG_PER_SUB, SEG_PER_SUB)],
                        out_hbm.at[c, pl.ds(s * SEG_PER_SUB, SEG_PER_SUB)])

    return k(H, idx1d, idx_tail)


def _combine(partials):
    def body(p_ref, o_ref):
        o_ref[...] = p_ref[0] + p_ref[1]

    return pl.pallas_call(
        body,
        out_shape=jax.ShapeDtypeStruct((NSEG, D), jnp.float32),
    )(partials)


def kernel(H, batch_idx):
    idx = batch_idx.astype(jnp.int32)
    idx_tail = idx[NFULL * CHUNK:]
    partials = _sc_partials(H, idx, idx_tail)
    return _combine(partials)
